# baseline (speedup 1.0000x reference)
import jax
import jax.numpy as jnp
from jax import lax
from jax.experimental import pallas as pl
from jax.experimental.pallas import tpu as pltpu
from jax.experimental.pallas import tpu_sc as plsc

NUM_ROIS = 1000
B, H, W, C = 2, 256, 256, 256
POOL = 7
NW = 32
RPW = 32
NROWS = 112
F32 = jnp.float32
BF16 = jnp.bfloat16
I32 = jnp.int32


def _body(table, roiflat, out, roiv, idx0, idx1, rows0, rows1, outv, sem0,
          sem1):
    wid = lax.axis_index("s") * 2 + lax.axis_index("c")
    pltpu.sync_copy(roiflat.at[pl.ds(wid * (RPW * 8), RPW * 8)], roiv)
    iota = lax.iota(I32, 16)
    iota_f = iota.astype(F32)
    lmask = iota < 14

    @pl.loop(0, RPW)
    def roi_loop(r):
        roi = wid * RPW + r

        @pl.when(roi < NUM_ROIS)
        def _():
            base = r * 8

            def splat_roi(col):
                return plsc.load_gather(
                    roiv, [jnp.full((16,), base + col, I32)])

            bf = splat_roi(0)
            y1 = splat_roi(1)
            x1 = splat_roi(2)
            y2 = splat_roi(3)
            x2 = splat_roi(4)
            bi = jnp.clip(bf.astype(I32), 0, B - 1)
            hscale = (y2 - y1) * float(H - 1) / 13.0
            wscale = (x2 - x1) * float(W - 1) / 13.0
            in_y = y1 * float(H - 1) + iota_f * hscale
            in_x = x1 * float(W - 1) + iota_f * wscale
            ty = jnp.clip(in_y.astype(I32), 0, H - 2)
            tx = jnp.clip(in_x.astype(I32), 0, W - 2)
            yl = in_y - ty.astype(F32)
            xl = in_x - tx.astype(F32)
            ybt = (bi * H + ty) * W

            def splat_lane(v, k):
                return v.at[jnp.full((16,), k, I32)].get(
                    mode="promise_in_bounds")

            def splat32(v, k):
                s = splat_lane(v, k)
                return plsc.pack(s, s, format=plsc.PackFormat.INTERLEAVED)

            bufs = [(idx0, rows0, sem0), (idx1, rows1, sem1)]

            def start_gather(oy):
                idxb, rows, sem = bufs[oy % 2]
                for iy in range(2):
                    ysp = splat_lane(ybt, 2 * oy + iy)
                    rowt = ysp + tx
                    rowb = rowt + W
                    g0 = iy * 4 * 14
                    plsc.store_scatter(idxb, [iota + g0], rowt, mask=lmask)
                    plsc.store_scatter(idxb, [iota + (g0 + 14)], rowt + 1,
                                       mask=lmask)
                    plsc.store_scatter(idxb, [iota + (g0 + 28)], rowb,
                                       mask=lmask)
                    plsc.store_scatter(idxb, [iota + (g0 + 42)], rowb + 1,
                                       mask=lmask)
                return pltpu.async_copy(table.at[idxb], rows, sem)

            dma = start_gather(0)
            for oy in range(POOL):
                rows = bufs[oy % 2][1]
                next_dma = start_gather(oy + 1) if oy + 1 < POOL else None
                dma.wait()
                dma = next_dma
                yl0 = splat32(yl, 2 * oy)
                yl1 = splat32(yl, 2 * oy + 1)
                for ox in range(POOL):
                    xs0 = splat32(xl, 2 * ox)
                    xs1 = splat32(xl, 2 * ox + 1)

                    @pl.loop(0, 8)
                    def chan_loop(k):
                        off = k * 16

                        def bil(iy, j, xs, ys):
                            b0 = 56 * iy
                            tl = plsc.bitcast(rows[b0 + j, pl.ds(off, 16)],
                                              BF16)
                            tr = plsc.bitcast(
                                rows[b0 + 14 + j, pl.ds(off, 16)], BF16)
                            bl = plsc.bitcast(
                                rows[b0 + 28 + j, pl.ds(off, 16)], BF16)
                            br = plsc.bitcast(
                                rows[b0 + 42 + j, pl.ds(off, 16)], BF16)
                            top = tl + (tr - tl) * xs
                            bot = bl + (br - bl) * xs
                            return top + (bot - top) * ys

                        v00 = bil(0, 2 * ox, xs0, yl0)
                        v01 = bil(0, 2 * ox + 1, xs1, yl0)
                        v10 = bil(1, 2 * ox, xs0, yl1)
                        v11 = bil(1, 2 * ox + 1, xs1, yl1)
                        vmax = jnp.maximum(
                            jnp.maximum(v00, v01), jnp.maximum(v10, v11))
                        outv[oy * POOL + ox, pl.ds(off, 16)] = plsc.bitcast(
                            vmax, I32)

            pltpu.sync_copy(outv, out.at[roi])


def _roialign_sc(table, roiflat):
    mesh = plsc.VectorSubcoreMesh(core_axis_name="c", subcore_axis_name="s")
    f = pl.kernel(
        _body,
        out_type=jax.ShapeDtypeStruct((NUM_ROIS, POOL * POOL, 128), I32),
        mesh=mesh,
        compiler_params=pltpu.CompilerParams(needs_layout_passes=False),
        scratch_types=[
            pltpu.VMEM((RPW * 8,), F32),
            pltpu.VMEM((NROWS,), I32),
            pltpu.VMEM((NROWS,), I32),
            pltpu.VMEM((NROWS, 128), I32),
            pltpu.VMEM((NROWS, 128), I32),
            pltpu.VMEM((POOL * POOL, 128), I32),
            pltpu.SemaphoreType.DMA,
            pltpu.SemaphoreType.DMA,
        ],
    )
    return f(table, roiflat)


def kernel(rois, feature_map, img_metas):
    del img_metas
    table = lax.bitcast_convert_type(
        feature_map.astype(BF16).reshape(B * H * W, 128, 2), I32)
    roiflat = jnp.pad(rois, ((0, NW * RPW - NUM_ROIS), (0, 3))).reshape(-1)
    out = _roialign_sc(table, roiflat)
    out = lax.bitcast_convert_type(out, BF16)
    return out.astype(F32).reshape(NUM_ROIS, POOL, POOL, C)


# speedup vs baseline: 1.5819x; 1.5819x over previous
import jax
import jax.numpy as jnp
from jax import lax
from jax.experimental import pallas as pl
from jax.experimental.pallas import tpu as pltpu
from jax.experimental.pallas import tpu_sc as plsc

NUM_ROIS = 1000
B, H, W, C = 2, 256, 256, 256
NR = B * H * W
POOL = 7
NW = 32
RPW = 32
NROWS = 112
CAST_RC = 32
F32 = jnp.float32
BF16 = jnp.bfloat16
I32 = jnp.int32

_MESH = dict(core_axis_name="c", subcore_axis_name="s")


def _wid():
    return lax.axis_index("s") * 2 + lax.axis_index("c")


def _cast_body(tbl, out, inb, outb):
    wid = _wid()
    rows_per_w = NR // NW

    @pl.loop(0, rows_per_w // CAST_RC)
    def chunk_loop(ci):
        base = wid * rows_per_w + ci * CAST_RC
        pltpu.sync_copy(tbl.at[pl.ds(base, CAST_RC)], inb)

        @pl.loop(0, CAST_RC)
        def row_loop(r):
            for w16 in range(8):
                u = inb[r, pl.ds(w16 * 16, 16)]
                v = inb[r, pl.ds(w16 * 16 + 128, 16)]
                outb[r, pl.ds(w16 * 16, 16)] = plsc.bitcast(
                    plsc.pack(u, v, format=plsc.PackFormat.INTERLEAVED), I32)

        pltpu.sync_copy(outb, out.at[pl.ds(base, CAST_RC)])


def _main_body(table, roiflat, out, roiv, idx0, idx1, rows0, rows1, outv,
               sem0, sem1):
    wid = _wid()
    base_roi = jnp.minimum(wid * RPW, NUM_ROIS - RPW)
    pltpu.sync_copy(roiflat.at[pl.ds(base_roi * 5, RPW * 5)], roiv)
    iota = lax.iota(I32, 16)
    iota_f = iota.astype(F32)
    lmask = iota < 14

    @pl.loop(0, RPW)
    def roi_loop(r):
        roi = wid * RPW + r

        @pl.when(roi < NUM_ROIS)
        def _():
            base = (roi - base_roi) * 5

            def splat_roi(col):
                return plsc.load_gather(
                    roiv, [jnp.full((16,), base + col, I32)])

            bf = splat_roi(0)
            y1 = splat_roi(1)
            x1 = splat_roi(2)
            y2 = splat_roi(3)
            x2 = splat_roi(4)
            bi = jnp.clip(bf.astype(I32), 0, B - 1)
            hscale = (y2 - y1) * float(H - 1) / 13.0
            wscale = (x2 - x1) * float(W - 1) / 13.0
            in_y = y1 * float(H - 1) + iota_f * hscale
            in_x = x1 * float(W - 1) + iota_f * wscale
            ty = jnp.clip(in_y.astype(I32), 0, H - 2)
            tx = jnp.clip(in_x.astype(I32), 0, W - 2)
            yl = in_y - ty.astype(F32)
            xl = in_x - tx.astype(F32)
            ybt = (bi * H + ty) * W

            def splat_lane(v, k):
                return v.at[jnp.full((16,), k, I32)].get(
                    mode="promise_in_bounds")

            def splat32(v, k):
                s = splat_lane(v, k)
                return plsc.pack(s, s, format=plsc.PackFormat.INTERLEAVED)

            bufs = [(idx0, rows0, sem0), (idx1, rows1, sem1)]

            def start_gather(oy):
                idxb, rows, sem = bufs[oy % 2]
                for iy in range(2):
                    ysp = splat_lane(ybt, 2 * oy + iy)
                    rowt = ysp + tx
                    rowb = rowt + W
                    g0 = iy * 4 * 14
                    plsc.store_scatter(idxb, [iota + g0], rowt, mask=lmask)
                    plsc.store_scatter(idxb, [iota + (g0 + 14)], rowt + 1,
                                       mask=lmask)
                    plsc.store_scatter(idxb, [iota + (g0 + 28)], rowb,
                                       mask=lmask)
                    plsc.store_scatter(idxb, [iota + (g0 + 42)], rowb + 1,
                                       mask=lmask)
                return pltpu.async_copy(table.at[idxb], rows, sem)

            dma = start_gather(0)
            for oy in range(POOL):
                rows = bufs[oy % 2][1]
                next_dma = start_gather(oy + 1) if oy + 1 < POOL else None
                dma.wait()
                dma = next_dma
                yl0 = splat32(yl, 2 * oy)
                yl1 = splat32(yl, 2 * oy + 1)
                for ox in range(POOL):
                    xs0 = splat32(xl, 2 * ox)
                    xs1 = splat32(xl, 2 * ox + 1)

                    @pl.loop(0, 8)
                    def chan_loop(k):
                        off = k * 16

                        def bil(iy, j, xs, ys):
                            b0 = 56 * iy
                            tl = plsc.bitcast(rows[b0 + j, pl.ds(off, 16)],
                                              BF16)
                            tr = plsc.bitcast(
                                rows[b0 + 14 + j, pl.ds(off, 16)], BF16)
                            bl = plsc.bitcast(
                                rows[b0 + 28 + j, pl.ds(off, 16)], BF16)
                            br = plsc.bitcast(
                                rows[b0 + 42 + j, pl.ds(off, 16)], BF16)
                            top = tl + (tr - tl) * xs
                            bot = bl + (br - bl) * xs
                            return top + (bot - top) * ys

                        v00 = bil(0, 2 * ox, xs0, yl0)
                        v01 = bil(0, 2 * ox + 1, xs1, yl0)
                        v10 = bil(1, 2 * ox, xs0, yl1)
                        v11 = bil(1, 2 * ox + 1, xs1, yl1)
                        vmax = jnp.maximum(
                            jnp.maximum(v00, v01), jnp.maximum(v10, v11))
                        lo, hi = plsc.unpack(
                            vmax, format=plsc.PackFormat.INTERLEAVED)
                        cell = oy * POOL + ox
                        outv[cell, pl.ds(off, 16)] = lo
                        outv[cell, pl.ds(off + 128, 16)] = hi

            pltpu.sync_copy(outv, out.at[roi])


def _roialign_sc(table_f32, roiflat):
    mesh = plsc.VectorSubcoreMesh(**_MESH)
    cp = pltpu.CompilerParams(needs_layout_passes=False)
    cast_k = pl.kernel(
        _cast_body,
        out_type=jax.ShapeDtypeStruct((NR, 128), I32),
        mesh=mesh,
        compiler_params=cp,
        scratch_types=[
            pltpu.VMEM((CAST_RC, 256), F32),
            pltpu.VMEM((CAST_RC, 128), I32),
        ],
    )
    table = cast_k(table_f32)
    main_k = pl.kernel(
        _main_body,
        out_type=jax.ShapeDtypeStruct((NUM_ROIS, POOL * POOL, C), F32),
        mesh=mesh,
        compiler_params=cp,
        scratch_types=[
            pltpu.VMEM((RPW * 5,), F32),
            pltpu.VMEM((NROWS,), I32),
            pltpu.VMEM((NROWS,), I32),
            pltpu.VMEM((NROWS, 128), I32),
            pltpu.VMEM((NROWS, 128), I32),
            pltpu.VMEM((POOL * POOL, C), F32),
            pltpu.SemaphoreType.DMA,
            pltpu.SemaphoreType.DMA,
        ],
    )
    return main_k(table, roiflat)


def kernel(rois, feature_map, img_metas):
    del img_metas
    out = _roialign_sc(feature_map.reshape(NR, C), rois.reshape(-1))
    return out.reshape(NUM_ROIS, POOL, POOL, C)


# speedup vs baseline: 1.9657x; 1.2426x over previous
import jax
import jax.numpy as jnp
from jax import lax
from jax.experimental import pallas as pl
from jax.experimental.pallas import tpu as pltpu
from jax.experimental.pallas import tpu_sc as plsc

NUM_ROIS = 1000
B, H, W, C = 2, 256, 256, 256
NR = B * H * W
POOL = 7
NW = 32
RPW = 32
NSEG = 56
CAST_RC = 32
F32 = jnp.float32
BF16 = jnp.bfloat16
I32 = jnp.int32

_MESH = dict(core_axis_name="c", subcore_axis_name="s")


def _wid():
    return lax.axis_index("s") * 2 + lax.axis_index("c")


def _cast_body(tbl, out, in0, in1, out0, out1, isem0, isem1, osem0, osem1):
    wid = _wid()
    rows_per_w = NR // NW
    nchunk = rows_per_w // CAST_RC
    ins = [(in0, isem0), (in1, isem1)]
    outs = [(out0, osem0), (out1, osem1)]

    RD = CAST_RC + 8

    def bases(ci):
        orig = wid * rows_per_w + ci * CAST_RC
        read = jnp.minimum(orig, NR - RD)
        return orig, read, orig - read

    def start_read(ci, p):
        inb, isem = ins[p]
        _, read, _ = bases(ci)
        return pltpu.async_copy(tbl.at[pl.ds(read, RD)], inb, isem)

    def pack_row(inb, j):
        def pk(w16):
            u = inb[j, pl.ds(w16 * 16, 16)]
            v = inb[j, pl.ds(w16 * 16 + 128, 16)]
            return plsc.bitcast(
                plsc.pack(u, v, format=plsc.PackFormat.INTERLEAVED), I32)
        return pk

    start_read(0, 0)

    @pl.loop(0, nchunk, step=2)
    def chunk_loop(ci0):
        for p in range(2):
            ci = ci0 + p
            inb, isem = ins[p]
            outb, osem = outs[p]
            orig, _, doff = bases(ci)

            @pl.when(ci + 1 < nchunk)
            def _():
                start_read(ci + 1, (p + 1) % 2)

            pltpu.make_async_copy(
                tbl.at[pl.ds(0, RD)], inb, isem).wait()

            @pl.when(ci >= 2)
            def _():
                pltpu.make_async_copy(
                    outb, out.at[pl.ds(0, CAST_RC)], osem).wait()

            pk0 = pack_row(inb, doff)
            pkN = pack_row(inb, jnp.minimum(doff + CAST_RC, RD - 1))
            for w16 in range(8):
                outb[0, pl.ds(w16 * 16, 16)] = pk0(w16)
                outb[CAST_RC - 1, pl.ds(w16 * 16 + 128, 16)] = pkN(w16)

            @pl.loop(1, CAST_RC)
            def row_loop(j):
                pkj = pack_row(inb, j + doff)
                for w16 in range(8):
                    w = pkj(w16)
                    outb[j, pl.ds(w16 * 16, 16)] = w
                    outb[j - 1, pl.ds(w16 * 16 + 128, 16)] = w

            pltpu.async_copy(outb, out.at[pl.ds(orig, CAST_RC)], osem)

    for p in range(2):
        outb, osem = outs[p]
        pltpu.make_async_copy(outb, out.at[pl.ds(0, CAST_RC)], osem).wait()


def _main_body(table, roiflat, out, roiv, idx0, idx1, rows0, rows1, outv,
               sem0, sem1):
    wid = _wid()
    base_roi = jnp.minimum(wid * RPW, NUM_ROIS - RPW)
    pltpu.sync_copy(roiflat.at[pl.ds(base_roi * 5, RPW * 5)], roiv)
    iota = lax.iota(I32, 16)
    iota_f = iota.astype(F32)
    lmask = iota < 14

    @pl.loop(0, RPW)
    def roi_loop(r):
        roi = wid * RPW + r

        @pl.when(roi < NUM_ROIS)
        def _():
            base = (roi - base_roi) * 5

            def splat_roi(col):
                return plsc.load_gather(
                    roiv, [jnp.full((16,), base + col, I32)])

            bf = splat_roi(0)
            y1 = splat_roi(1)
            x1 = splat_roi(2)
            y2 = splat_roi(3)
            x2 = splat_roi(4)
            bi = jnp.clip(bf.astype(I32), 0, B - 1)
            hscale = (y2 - y1) * float(H - 1) / 13.0
            wscale = (x2 - x1) * float(W - 1) / 13.0
            in_y = y1 * float(H - 1) + iota_f * hscale
            in_x = x1 * float(W - 1) + iota_f * wscale
            ty = jnp.clip(in_y.astype(I32), 0, H - 2)
            tx = jnp.clip(in_x.astype(I32), 0, W - 2)
            yl = in_y - ty.astype(F32)
            xl = in_x - tx.astype(F32)
            ybt = (bi * H + ty) * W

            def splat_lane(v, k):
                return v.at[jnp.full((16,), k, I32)].get(
                    mode="promise_in_bounds")

            def splat32(v, k):
                s = splat_lane(v, k)
                return plsc.pack(s, s, format=plsc.PackFormat.INTERLEAVED)

            bufs = [(idx0, rows0, sem0), (idx1, rows1, sem1)]

            def start_gather(oy):
                idxb, rows, sem = bufs[oy % 2]
                for iy in range(2):
                    ysp = splat_lane(ybt, 2 * oy + iy)
                    rowt = ysp + tx
                    g0 = iy * 2 * 14
                    plsc.store_scatter(idxb, [iota + g0], rowt, mask=lmask)
                    plsc.store_scatter(idxb, [iota + (g0 + 14)], rowt + W,
                                       mask=lmask)
                return pltpu.async_copy(table.at[idxb], rows, sem)

            dma = start_gather(0)
            for oy in range(POOL):
                rows = bufs[oy % 2][1]
                next_dma = start_gather(oy + 1) if oy + 1 < POOL else None
                dma.wait()
                dma = next_dma
                yl0 = splat32(yl, 2 * oy)
                yl1 = splat32(yl, 2 * oy + 1)
                for ox in range(POOL):
                    xs0 = splat32(xl, 2 * ox)
                    xs1 = splat32(xl, 2 * ox + 1)

                    @pl.loop(0, 8)
                    def chan_loop(k):
                        off = k * 16

                        def bil(iy, j, xs, ys):
                            pt = 28 * iy + j
                            tl = plsc.bitcast(rows[pt, pl.ds(off, 16)], BF16)
                            tr = plsc.bitcast(rows[pt, pl.ds(off + 128, 16)],
                                              BF16)
                            bl = plsc.bitcast(rows[pt + 14, pl.ds(off, 16)],
                                              BF16)
                            br = plsc.bitcast(
                                rows[pt + 14, pl.ds(off + 128, 16)], BF16)
                            top = tl + (tr - tl) * xs
                            bot = bl + (br - bl) * xs
                            return top + (bot - top) * ys

                        v00 = bil(0, 2 * ox, xs0, yl0)
                        v01 = bil(0, 2 * ox + 1, xs1, yl0)
                        v10 = bil(1, 2 * ox, xs0, yl1)
                        v11 = bil(1, 2 * ox + 1, xs1, yl1)
                        vmax = jnp.maximum(
                            jnp.maximum(v00, v01), jnp.maximum(v10, v11))
                        lo, hi = plsc.unpack(
                            vmax, format=plsc.PackFormat.INTERLEAVED)
                        cell = oy * POOL + ox
                        outv[cell, pl.ds(off, 16)] = lo
                        outv[cell, pl.ds(off + 128, 16)] = hi

            pltpu.sync_copy(outv, out.at[roi])


def _roialign_sc(table_f32, roiflat):
    mesh = plsc.VectorSubcoreMesh(**_MESH)
    cp = pltpu.CompilerParams(needs_layout_passes=False)
    cast_k = pl.kernel(
        _cast_body,
        out_type=jax.ShapeDtypeStruct((NR, 256), I32),
        mesh=mesh,
        compiler_params=cp,
        scratch_types=[
            pltpu.VMEM((CAST_RC + 8, 256), F32),
            pltpu.VMEM((CAST_RC + 8, 256), F32),
            pltpu.VMEM((CAST_RC, 256), I32),
            pltpu.VMEM((CAST_RC, 256), I32),
            pltpu.SemaphoreType.DMA,
            pltpu.SemaphoreType.DMA,
            pltpu.SemaphoreType.DMA,
            pltpu.SemaphoreType.DMA,
        ],
    )
    table = cast_k(table_f32)
    main_k = pl.kernel(
        _main_body,
        out_type=jax.ShapeDtypeStruct((NUM_ROIS, POOL * POOL, C), F32),
        mesh=mesh,
        compiler_params=cp,
        scratch_types=[
            pltpu.VMEM((RPW * 5,), F32),
            pltpu.VMEM((NSEG,), I32),
            pltpu.VMEM((NSEG,), I32),
            pltpu.VMEM((NSEG, 256), I32),
            pltpu.VMEM((NSEG, 256), I32),
            pltpu.VMEM((POOL * POOL, C), F32),
            pltpu.SemaphoreType.DMA,
            pltpu.SemaphoreType.DMA,
        ],
    )
    return main_k(table, roiflat)


def kernel(rois, feature_map, img_metas):
    del img_metas
    out = _roialign_sc(feature_map.reshape(NR, C), rois.reshape(-1))
    return out.reshape(NUM_ROIS, POOL, POOL, C)


# speedup vs baseline: 2.1572x; 1.0974x over previous
import jax
import jax.numpy as jnp
from jax import lax
from jax.experimental import pallas as pl
from jax.experimental.pallas import tpu as pltpu
from jax.experimental.pallas import tpu_sc as plsc

NUM_ROIS = 1000
B, H, W, C = 2, 256, 256, 256
NR = B * H * W
POOL = 7
NW = 32
RPW = 32
NSEG = 56
CAST_RC = 32
F32 = jnp.float32
BF16 = jnp.bfloat16
I32 = jnp.int32

_MESH = dict(core_axis_name="c", subcore_axis_name="s")


def _wid():
    return lax.axis_index("s") * 2 + lax.axis_index("c")


def _cast_body(tbl, out, in0, in1, out0, out1, isem0, isem1, osem0, osem1):
    wid = _wid()
    rows_per_w = NR // NW
    nchunk = rows_per_w // CAST_RC
    ins = [(in0, isem0), (in1, isem1)]
    outs = [(out0, osem0), (out1, osem1)]

    RD = CAST_RC + 8

    def bases(ci):
        orig = wid * rows_per_w + ci * CAST_RC
        read = jnp.minimum(orig, NR - RD)
        return orig, read, orig - read

    def start_read(ci, p):
        inb, isem = ins[p]
        _, read, _ = bases(ci)
        return pltpu.async_copy(tbl.at[pl.ds(read, RD)], inb, isem)

    def pack_row(inb, j):
        def pk(w16):
            u = inb[j, pl.ds(w16 * 16, 16)]
            v = inb[j, pl.ds(w16 * 16 + 128, 16)]
            return plsc.bitcast(
                plsc.pack(u, v, format=plsc.PackFormat.INTERLEAVED), I32)
        return pk

    start_read(0, 0)

    @pl.loop(0, nchunk, step=2)
    def chunk_loop(ci0):
        for p in range(2):
            ci = ci0 + p
            inb, isem = ins[p]
            outb, osem = outs[p]
            orig, _, doff = bases(ci)

            @pl.when(ci + 1 < nchunk)
            def _():
                start_read(ci + 1, (p + 1) % 2)

            pltpu.make_async_copy(
                tbl.at[pl.ds(0, RD)], inb, isem).wait()

            @pl.when(ci >= 2)
            def _():
                pltpu.make_async_copy(
                    outb, out.at[pl.ds(0, CAST_RC)], osem).wait()

            pk0 = pack_row(inb, doff)
            pkN = pack_row(inb, jnp.minimum(doff + CAST_RC, RD - 1))
            for w16 in range(8):
                outb[0, pl.ds(w16 * 16, 16)] = pk0(w16)
                outb[CAST_RC - 1, pl.ds(w16 * 16 + 128, 16)] = pkN(w16)

            @pl.loop(1, CAST_RC)
            def row_loop(j):
                pkj = pack_row(inb, j + doff)
                for w16 in range(8):
                    w = pkj(w16)
                    outb[j, pl.ds(w16 * 16, 16)] = w
                    outb[j - 1, pl.ds(w16 * 16 + 128, 16)] = w

            pltpu.async_copy(outb, out.at[pl.ds(orig, CAST_RC)], osem)

    for p in range(2):
        outb, osem = outs[p]
        pltpu.make_async_copy(outb, out.at[pl.ds(0, CAST_RC)], osem).wait()


def _main_body(table, roiflat, out, roiv, idx0, idx1, rows0, rows1, outv0,
               outv1, sem0, sem1, osem0, osem1):
    wid = _wid()
    base_roi = jnp.minimum(wid * RPW, NUM_ROIS - RPW)
    pltpu.sync_copy(roiflat.at[pl.ds(base_roi * 5, RPW * 5)], roiv)
    iota = lax.iota(I32, 16)
    iota_f = iota.astype(F32)
    lmask = iota < 14

    outvs = [(outv0, osem0), (outv1, osem1)]

    @pl.loop(0, RPW, step=2)
    def roi_loop(r0):
      for p in range(2):
        r = r0 + p
        roi = wid * RPW + r
        outv, osem = outvs[p]

        @pl.when(roi < NUM_ROIS)
        def _():
            @pl.when(r >= 2)
            def _():
                pltpu.make_async_copy(outv, out.at[0], osem).wait()

            base = (roi - base_roi) * 5

            def splat_roi(col):
                return plsc.load_gather(
                    roiv, [jnp.full((16,), base + col, I32)])

            bf = splat_roi(0)
            y1 = splat_roi(1)
            x1 = splat_roi(2)
            y2 = splat_roi(3)
            x2 = splat_roi(4)
            bi = jnp.clip(bf.astype(I32), 0, B - 1)
            hscale = (y2 - y1) * float(H - 1) / 13.0
            wscale = (x2 - x1) * float(W - 1) / 13.0
            in_y = y1 * float(H - 1) + iota_f * hscale
            in_x = x1 * float(W - 1) + iota_f * wscale
            ty = jnp.clip(in_y.astype(I32), 0, H - 2)
            tx = jnp.clip(in_x.astype(I32), 0, W - 2)
            yl = in_y - ty.astype(F32)
            xl = in_x - tx.astype(F32)
            ybt = (bi * H + ty) * W

            def splat_lane(v, k):
                return v.at[jnp.full((16,), k, I32)].get(
                    mode="promise_in_bounds")

            def splat32(v, k):
                s = splat_lane(v, k)
                return plsc.pack(s, s, format=plsc.PackFormat.INTERLEAVED)

            bufs = [(idx0, rows0, sem0), (idx1, rows1, sem1)]

            def start_gather(oy, p):
                idxb, rows, sem = bufs[p]
                for iy in range(2):
                    ysp = splat_lane(ybt, 2 * oy + iy)
                    rowt = ysp + tx
                    g0 = iy * 2 * 14
                    plsc.store_scatter(idxb, [iota + g0], rowt, mask=lmask)
                    plsc.store_scatter(idxb, [iota + (g0 + 14)], rowt + W,
                                       mask=lmask)
                return pltpu.async_copy(table.at[idxb], rows, sem)

            def wait_gather(p):
                idxb, rows, sem = bufs[p]
                pltpu.make_async_copy(table.at[idxb], rows, sem).wait()

            def compute_chunk(oy, rows):
                yl0 = splat32(yl, 2 * oy)
                yl1 = splat32(yl, 2 * oy + 1)
                @plsc.parallel_loop(0, POOL)
                def ox_loop(ox):
                    xs0 = splat32(xl, 2 * ox)
                    xs1 = splat32(xl, 2 * ox + 1)
                    pt0 = 2 * ox
                    pt1 = 28 + 2 * ox
                    cell = oy * POOL + ox

                    for k in range(8):
                        off = k * 16

                        def bil(pt, j1, xs, ys):
                            tl = plsc.bitcast(rows[pt + j1, pl.ds(off, 16)],
                                              BF16)
                            tr = plsc.bitcast(
                                rows[pt + j1, pl.ds(off + 128, 16)], BF16)
                            bl = plsc.bitcast(
                                rows[pt + j1 + 14, pl.ds(off, 16)], BF16)
                            br = plsc.bitcast(
                                rows[pt + j1 + 14, pl.ds(off + 128, 16)],
                                BF16)
                            top = tl + (tr - tl) * xs
                            bot = bl + (br - bl) * xs
                            return top + (bot - top) * ys

                        v00 = bil(pt0, 0, xs0, yl0)
                        v01 = bil(pt0, 1, xs1, yl0)
                        v10 = bil(pt1, 0, xs0, yl1)
                        v11 = bil(pt1, 1, xs1, yl1)
                        vmax = jnp.maximum(
                            jnp.maximum(v00, v01), jnp.maximum(v10, v11))
                        lo, hi = plsc.unpack(
                            vmax, format=plsc.PackFormat.INTERLEAVED)
                        outv[cell, pl.ds(off, 16)] = lo
                        outv[cell, pl.ds(off + 128, 16)] = hi

            start_gather(0, 0)

            @pl.loop(0, POOL - 1, step=2)
            def oy_loop(oy0):
                for p in range(2):
                    oy = oy0 + p
                    start_gather(oy + 1, (p + 1) % 2)
                    wait_gather(p)
                    compute_chunk(oy, bufs[p][1])

            wait_gather(0)
            compute_chunk(POOL - 1, bufs[0][1])

            pltpu.async_copy(outv, out.at[roi], osem)

    for p in range(2):
        outv, osem = outvs[p]
        pltpu.make_async_copy(outv, out.at[0], osem).wait()


def _roialign_sc(table_f32, roiflat):
    mesh = plsc.VectorSubcoreMesh(**_MESH)
    cp = pltpu.CompilerParams(needs_layout_passes=False)
    cast_k = pl.kernel(
        _cast_body,
        out_type=jax.ShapeDtypeStruct((NR, 256), I32),
        mesh=mesh,
        compiler_params=cp,
        scratch_types=[
            pltpu.VMEM((CAST_RC + 8, 256), F32),
            pltpu.VMEM((CAST_RC + 8, 256), F32),
            pltpu.VMEM((CAST_RC, 256), I32),
            pltpu.VMEM((CAST_RC, 256), I32),
            pltpu.SemaphoreType.DMA,
            pltpu.SemaphoreType.DMA,
            pltpu.SemaphoreType.DMA,
            pltpu.SemaphoreType.DMA,
        ],
    )
    table = cast_k(table_f32)
    main_k = pl.kernel(
        _main_body,
        out_type=jax.ShapeDtypeStruct((NUM_ROIS, POOL * POOL, C), F32),
        mesh=mesh,
        compiler_params=cp,
        scratch_types=[
            pltpu.VMEM((RPW * 5,), F32),
            pltpu.VMEM((NSEG,), I32),
            pltpu.VMEM((NSEG,), I32),
            pltpu.VMEM((NSEG, 256), I32),
            pltpu.VMEM((NSEG, 256), I32),
            pltpu.VMEM((POOL * POOL, C), F32),
            pltpu.VMEM((POOL * POOL, C), F32),
            pltpu.SemaphoreType.DMA,
            pltpu.SemaphoreType.DMA,
            pltpu.SemaphoreType.DMA,
            pltpu.SemaphoreType.DMA,
        ],
    )
    return main_k(table, roiflat)


def kernel(rois, feature_map, img_metas):
    del img_metas
    out = _roialign_sc(feature_map.reshape(NR, C), rois.reshape(-1))
    return out.reshape(NUM_ROIS, POOL, POOL, C)
